# Initial kernel scaffold; baseline (speedup 1.0000x reference)
import jax
import jax.numpy as jnp
from jax.experimental import pallas as pl


def kernel(x, w1, b1, w2, b2, w3, wd, b3):
    raise NotImplementedError("write your pallas kernel here")


# speedup vs baseline: 2.9340x; 2.9340x over previous
import functools

import jax
import jax.numpy as jnp
import numpy as np
from jax.experimental import pallas as pl
from jax.experimental.pallas import tpu as pltpu


def _bottleneck_body(x_ref, w1_ref, b1_ref, w2_ref, b2_ref, w3_ref, b3_ref,
                     p_ref, o_ref, *, H, W):
    HW = H * W
    x2 = x_ref[0]

    h1 = jnp.dot(w1_ref[...], x2, preferred_element_type=jnp.float32)
    h1 = jnp.maximum(h1 + b1_ref[...], 0.0)

    pad = W + 1
    hp = jnp.pad(h1, ((0, 0), (pad, pad)))
    wmod = jax.lax.broadcasted_iota(jnp.int32, (1, HW), 1) % W
    m_left = (wmod >= 1).astype(jnp.float32)
    m_right = (wmod <= W - 2).astype(jnp.float32)
    patches = []
    for dy in range(3):
        for dx in range(3):
            s = (dy - 1) * W + (dx - 1)
            pt = hp[:, pad + s: pad + s + HW]
            if dx == 0:
                pt = pt * m_left
            elif dx == 2:
                pt = pt * m_right
            patches.append(pt)
    pat = jnp.concatenate(patches, axis=0)
    h2 = jnp.dot(w2_ref[...], pat, preferred_element_type=jnp.float32)
    h2 = jnp.maximum(h2 + b2_ref[...], 0.0)

    h2p = jnp.dot(h2, p_ref[...], preferred_element_type=jnp.float32)
    xp = jnp.dot(x2, p_ref[...], preferred_element_type=jnp.float32)
    cat = jnp.concatenate([h2p, xp], axis=0)
    y = jnp.dot(w3_ref[...], cat, preferred_element_type=jnp.float32)
    y = jnp.maximum(y + b3_ref[...], 0.0)
    o_ref[0] = y


def kernel(x, w1, b1, w2, b2, w3, wd, b3):
    N, Cin, H, W = x.shape
    C2 = w1.shape[1]
    Cout = w3.shape[1]
    K = 2
    Ho, Wo = H // K, W // K
    HW, HoWo = H * W, Ho * Wo

    w1t = w1.T
    w2t = jnp.concatenate([w2[t].T for t in range(9)], axis=1)
    w3t = jnp.concatenate([w3.T, wd.T], axis=1)
    b1c, b2c, b3c = b1.T, b2.T, b3.T

    r = np.arange(HW)
    j = (r // W // K) * Wo + (r % W) // K
    p_np = np.zeros((HW, HoWo), np.float32)
    p_np[r, j] = 1.0
    pmat = jnp.asarray(p_np)

    x3 = x.reshape(N, Cin, HW)
    out = pl.pallas_call(
        functools.partial(_bottleneck_body, H=H, W=W),
        out_shape=jax.ShapeDtypeStruct((N, Cout, HoWo), jnp.float32),
        grid=(N,),
        in_specs=[
            pl.BlockSpec((1, Cin, HW), lambda i: (i, 0, 0)),
            pl.BlockSpec((C2, Cin), lambda i: (0, 0)),
            pl.BlockSpec((C2, 1), lambda i: (0, 0)),
            pl.BlockSpec((C2, 9 * C2), lambda i: (0, 0)),
            pl.BlockSpec((C2, 1), lambda i: (0, 0)),
            pl.BlockSpec((Cout, C2 + Cin), lambda i: (0, 0)),
            pl.BlockSpec((Cout, 1), lambda i: (0, 0)),
            pl.BlockSpec((HW, HoWo), lambda i: (0, 0)),
        ],
        out_specs=pl.BlockSpec((1, Cout, HoWo), lambda i: (i, 0, 0)),
        compiler_params=pltpu.CompilerParams(
            dimension_semantics=("parallel",),
            vmem_limit_bytes=64 * 1024 * 1024,
        ),
    )(x3, w1t, b1c, w2t, b2c, w3t, b3c, pmat)
    return out.reshape(N, Cout, Ho, Wo)


# speedup vs baseline: 3.1019x; 1.0572x over previous
import functools

import jax
import jax.numpy as jnp
import numpy as np
from jax.experimental import pallas as pl
from jax.experimental.pallas import tpu as pltpu


def _bottleneck_body(x_ref, w1_ref, b1_ref, w2_ref, b2_ref, w3_ref, b3_ref,
                     p_ref, o_ref, *, H, W, G):
    HW = H * W
    pad = W + 1
    wmod = jax.lax.broadcasted_iota(jnp.int32, (1, HW), 1) % W
    m_left = (wmod >= 1).astype(jnp.float32)
    m_right = (wmod <= W - 2).astype(jnp.float32)

    for g in range(G):
        x2 = x_ref[g]

        h1 = jnp.dot(w1_ref[...], x2, preferred_element_type=jnp.float32)
        h1 = jnp.maximum(h1 + b1_ref[...], 0.0)

        hp = jnp.pad(h1, ((0, 0), (pad, pad)))
        patches = []
        for dy in range(3):
            for dx in range(3):
                s = (dy - 1) * W + (dx - 1)
                pt = hp[:, pad + s: pad + s + HW]
                if dx == 0:
                    pt = pt * m_left
                elif dx == 2:
                    pt = pt * m_right
                patches.append(pt)
        pat = jnp.concatenate(patches, axis=0)
        h2 = jnp.dot(w2_ref[...], pat, preferred_element_type=jnp.float32)
        h2 = jnp.maximum(h2 + b2_ref[...], 0.0)

        h2p = jnp.dot(h2, p_ref[...], preferred_element_type=jnp.float32)
        xp = jnp.dot(x2, p_ref[...], preferred_element_type=jnp.float32)
        cat = jnp.concatenate([h2p, xp], axis=0)
        y = jnp.dot(w3_ref[...], cat, preferred_element_type=jnp.float32)
        y = jnp.maximum(y + b3_ref[...], 0.0)
        o_ref[g] = y


def kernel(x, w1, b1, w2, b2, w3, wd, b3):
    N, Cin, H, W = x.shape
    C2 = w1.shape[1]
    Cout = w3.shape[1]
    K = 2
    Ho, Wo = H // K, W // K
    HW, HoWo = H * W, Ho * Wo

    w1t = w1.T
    w2t = jnp.concatenate([w2[t].T for t in range(9)], axis=1)
    w3t = jnp.concatenate([w3.T, wd.T], axis=1)
    b1c, b2c, b3c = b1.T, b2.T, b3.T

    r = np.arange(HW)
    j = (r // W // K) * Wo + (r % W) // K
    p_np = np.zeros((HW, HoWo), np.float32)
    p_np[r, j] = 1.0
    pmat = jnp.asarray(p_np)

    G = 4
    x3 = x.reshape(N, Cin, HW)
    out = pl.pallas_call(
        functools.partial(_bottleneck_body, H=H, W=W, G=G),
        out_shape=jax.ShapeDtypeStruct((N, Cout, HoWo), jnp.float32),
        grid=(N // G,),
        in_specs=[
            pl.BlockSpec((G, Cin, HW), lambda i: (i, 0, 0)),
            pl.BlockSpec((C2, Cin), lambda i: (0, 0)),
            pl.BlockSpec((C2, 1), lambda i: (0, 0)),
            pl.BlockSpec((C2, 9 * C2), lambda i: (0, 0)),
            pl.BlockSpec((C2, 1), lambda i: (0, 0)),
            pl.BlockSpec((Cout, C2 + Cin), lambda i: (0, 0)),
            pl.BlockSpec((Cout, 1), lambda i: (0, 0)),
            pl.BlockSpec((HW, HoWo), lambda i: (0, 0)),
        ],
        out_specs=pl.BlockSpec((G, Cout, HoWo), lambda i: (i, 0, 0)),
        compiler_params=pltpu.CompilerParams(
            dimension_semantics=("parallel",),
            vmem_limit_bytes=64 * 1024 * 1024,
        ),
    )(x3, w1t, b1c, w2t, b2c, w3t, b3c, pmat)
    return out.reshape(N, Cout, Ho, Wo)
